# baseline (speedup 1.0000x reference)
import functools

import jax
import jax.numpy as jnp
from jax import lax
from jax.experimental import pallas as pl
from jax.experimental.pallas import tpu as pltpu
from jax.experimental.pallas import tpu_sc as plsc

D_MODEL = 128
MAX_LEN = 1000

_NUM_CORES = 2
_NUM_SUBCORES = 16
_NUM_WORKERS = _NUM_CORES * _NUM_SUBCORES
_LANES = 16
_REPLICAS = 32

_CHUNK = 128


def _sc_body(x_hbm, ts_hbm, pe_hbm, out_hbm, idx_all, xbuf, perows,
             sem_in, sem_wb, *, chunks_per_worker):
    wid = lax.axis_index("s") * _NUM_CORES + lax.axis_index("c")
    idx_row0 = wid * chunks_per_worker
    row0 = idx_row0 * _CHUNK
    pe_copy = pe_hbm.at[lax.rem(wid, _REPLICAS)]

    pltpu.sync_copy(ts_hbm.at[pl.ds(idx_row0, chunks_per_worker)], idx_all)

    def clamp_row(c, carry):
        for j in range(_CHUNK // _LANES):
            s = pl.ds(j * _LANES, _LANES)
            idx_all[c, s] = jnp.minimum(jnp.maximum(idx_all[c, s], 0),
                                        MAX_LEN - 1)
        return carry

    lax.fori_loop(0, chunks_per_worker, clamp_row, 0, unroll=False)

    def start_fetch(c, slot):
        pltpu.async_copy(pe_copy.at[idx_all.at[c]], perows.at[slot],
                         sem_in.at[slot])
        pltpu.async_copy(x_hbm.at[pl.ds(row0 + c * _CHUNK, _CHUNK)],
                         xbuf.at[slot], sem_in.at[slot])

    def wait_fetch(c, slot):
        pltpu.make_async_copy(pe_copy.at[idx_all.at[c]], perows.at[slot],
                              sem_in.at[slot]).wait()
        pltpu.make_async_copy(x_hbm.at[pl.ds(row0 + c * _CHUNK, _CHUNK)],
                              xbuf.at[slot], sem_in.at[slot]).wait()

    def wait_wb(c, slot):
        pltpu.make_async_copy(xbuf.at[slot],
                              out_hbm.at[pl.ds(row0 + c * _CHUNK, _CHUNK)],
                              sem_wb.at[slot]).wait()

    start_fetch(0, 0)

    def pair_body(g, carry):
        for slot in range(2):
            c = 2 * g + slot
            other = 1 - slot

            @pl.when(c + 1 < chunks_per_worker)
            def _():
                @pl.when(c >= 1)
                def _():
                    wait_wb(c - 1, other)
                start_fetch(c + 1, other)

            wait_fetch(c, slot)

            def add_row(r, carry2, slot=slot):
                for j in range(D_MODEL // _LANES):
                    s = pl.ds(j * _LANES, _LANES)
                    xbuf[slot, r, s] = xbuf[slot, r, s] + perows[slot, r, s]
                return carry2

            lax.fori_loop(0, _CHUNK, add_row, 0, unroll=False)

            pltpu.async_copy(xbuf.at[slot],
                             out_hbm.at[pl.ds(row0 + c * _CHUNK, _CHUNK)],
                             sem_wb.at[slot])
        return carry

    lax.fori_loop(0, chunks_per_worker // 2, pair_body, 0, unroll=False)
    wait_wb(chunks_per_worker - 2, 0)
    wait_wb(chunks_per_worker - 1, 1)


def kernel(x, timestamps, pe):
    batch, seq, d = x.shape
    n = batch * seq
    assert d == D_MODEL and n % (_NUM_WORKERS * _CHUNK) == 0
    chunks_per_worker = n // (_NUM_WORKERS * _CHUNK)
    assert chunks_per_worker >= 2 and chunks_per_worker % 2 == 0

    x2 = x.reshape(n, d)
    ts2 = timestamps.astype(jnp.int32).reshape(n // _CHUNK, _CHUNK)
    pe_rep = jnp.broadcast_to(pe, (_REPLICAS,) + pe.shape)

    mesh = plsc.VectorSubcoreMesh(core_axis_name="c", subcore_axis_name="s")
    body = functools.partial(_sc_body, chunks_per_worker=chunks_per_worker)
    out = pl.kernel(
        body,
        out_type=jax.ShapeDtypeStruct((n, d), jnp.float32),
        mesh=mesh,
        scratch_types=[
            pltpu.VMEM((chunks_per_worker, _CHUNK), jnp.int32),
            pltpu.VMEM((2, _CHUNK, D_MODEL), jnp.float32),
            pltpu.VMEM((2, _CHUNK, D_MODEL), jnp.float32),
            pltpu.SemaphoreType.DMA((2,)),
            pltpu.SemaphoreType.DMA((2,)),
        ],
    )(x2, ts2, pe_rep)
    return out.reshape(batch, seq, d)


# speedup vs baseline: 1.0021x; 1.0021x over previous
import functools

import jax
import jax.numpy as jnp
from jax import lax
from jax.experimental import pallas as pl
from jax.experimental.pallas import tpu as pltpu
from jax.experimental.pallas import tpu_sc as plsc

D_MODEL = 128
MAX_LEN = 1000

_NUM_CORES = 2
_NUM_SUBCORES = 16
_NUM_WORKERS = _NUM_CORES * _NUM_SUBCORES
_LANES = 16
_REPLICAS = 8

_CHUNK = 128


def _sc_body(x_hbm, ts_hbm, pe_hbm, out_hbm, idx_all, xbuf, perows,
             sem_in, sem_wb, *, chunks_per_worker):
    wid = lax.axis_index("s") * _NUM_CORES + lax.axis_index("c")
    idx_row0 = wid * chunks_per_worker
    row0 = idx_row0 * _CHUNK
    pe_copy = pe_hbm.at[lax.rem(wid, _REPLICAS)]

    pltpu.sync_copy(ts_hbm.at[pl.ds(idx_row0, chunks_per_worker)], idx_all)

    def clamp_row(c, carry):
        for j in range(_CHUNK // _LANES):
            s = pl.ds(j * _LANES, _LANES)
            idx_all[c, s] = jnp.minimum(jnp.maximum(idx_all[c, s], 0),
                                        MAX_LEN - 1)
        return carry

    lax.fori_loop(0, chunks_per_worker, clamp_row, 0, unroll=False)

    half = _CHUNK // 2

    def start_fetch(c, slot):
        pltpu.async_copy(pe_copy.at[idx_all.at[c, pl.ds(0, half)]],
                         perows.at[slot, pl.ds(0, half)], sem_in.at[slot])
        pltpu.async_copy(pe_copy.at[idx_all.at[c, pl.ds(half, half)]],
                         perows.at[slot, pl.ds(half, half)], sem_in.at[slot])
        pltpu.async_copy(x_hbm.at[pl.ds(row0 + c * _CHUNK, _CHUNK)],
                         xbuf.at[slot], sem_in.at[slot])

    def wait_fetch(c, slot):
        pltpu.make_async_copy(pe_copy.at[idx_all.at[c, pl.ds(0, half)]],
                              perows.at[slot, pl.ds(0, half)],
                              sem_in.at[slot]).wait()
        pltpu.make_async_copy(pe_copy.at[idx_all.at[c, pl.ds(half, half)]],
                              perows.at[slot, pl.ds(half, half)],
                              sem_in.at[slot]).wait()
        pltpu.make_async_copy(x_hbm.at[pl.ds(row0 + c * _CHUNK, _CHUNK)],
                              xbuf.at[slot], sem_in.at[slot]).wait()

    def wait_wb(c, slot):
        pltpu.make_async_copy(xbuf.at[slot],
                              out_hbm.at[pl.ds(row0 + c * _CHUNK, _CHUNK)],
                              sem_wb.at[slot]).wait()

    start_fetch(0, 0)

    def pair_body(g, carry):
        for slot in range(2):
            c = 2 * g + slot
            other = 1 - slot

            @pl.when(c + 1 < chunks_per_worker)
            def _():
                @pl.when(c >= 1)
                def _():
                    wait_wb(c - 1, other)
                start_fetch(c + 1, other)

            wait_fetch(c, slot)

            def add_row(r, carry2, slot=slot):
                for j in range(D_MODEL // _LANES):
                    s = pl.ds(j * _LANES, _LANES)
                    xbuf[slot, r, s] = xbuf[slot, r, s] + perows[slot, r, s]
                return carry2

            lax.fori_loop(0, _CHUNK, add_row, 0, unroll=False)

            pltpu.async_copy(xbuf.at[slot],
                             out_hbm.at[pl.ds(row0 + c * _CHUNK, _CHUNK)],
                             sem_wb.at[slot])
        return carry

    lax.fori_loop(0, chunks_per_worker // 2, pair_body, 0, unroll=False)
    wait_wb(chunks_per_worker - 2, 0)
    wait_wb(chunks_per_worker - 1, 1)


def kernel(x, timestamps, pe):
    batch, seq, d = x.shape
    n = batch * seq
    assert d == D_MODEL and n % (_NUM_WORKERS * _CHUNK) == 0
    chunks_per_worker = n // (_NUM_WORKERS * _CHUNK)
    assert chunks_per_worker >= 2 and chunks_per_worker % 2 == 0

    x2 = x.reshape(n, d)
    ts2 = timestamps.astype(jnp.int32).reshape(n // _CHUNK, _CHUNK)
    pe_rep = jnp.broadcast_to(pe, (_REPLICAS,) + pe.shape)

    mesh = plsc.VectorSubcoreMesh(core_axis_name="c", subcore_axis_name="s")
    body = functools.partial(_sc_body, chunks_per_worker=chunks_per_worker)
    out = pl.kernel(
        body,
        out_type=jax.ShapeDtypeStruct((n, d), jnp.float32),
        mesh=mesh,
        scratch_types=[
            pltpu.VMEM((chunks_per_worker, _CHUNK), jnp.int32),
            pltpu.VMEM((2, _CHUNK, D_MODEL), jnp.float32),
            pltpu.VMEM((2, _CHUNK, D_MODEL), jnp.float32),
            pltpu.SemaphoreType.DMA((2,)),
            pltpu.SemaphoreType.DMA((2,)),
        ],
    )(x2, ts2, pe_rep)
    return out.reshape(batch, seq, d)


# speedup vs baseline: 1.0295x; 1.0274x over previous
import functools

import jax
import jax.numpy as jnp
from jax import lax
from jax.experimental import pallas as pl
from jax.experimental.pallas import tpu as pltpu
from jax.experimental.pallas import tpu_sc as plsc

D_MODEL = 128
MAX_LEN = 1000

_NUM_CORES = 2
_NUM_SUBCORES = 16
_NUM_WORKERS = _NUM_CORES * _NUM_SUBCORES
_LANES = 16
_REPLICAS = 8
_DEPTH = 3

_CHUNK = 128


def _sc_body(x_hbm, ts_hbm, pe_hbm, out_hbm, idx_all, xbuf, perows,
             sem_in, sem_wb, *, chunks_per_worker):
    wid = lax.axis_index("s") * _NUM_CORES + lax.axis_index("c")
    idx_row0 = wid * chunks_per_worker
    row0 = idx_row0 * _CHUNK
    pe_copy = pe_hbm.at[lax.rem(wid, _REPLICAS)]

    pltpu.sync_copy(ts_hbm.at[pl.ds(idx_row0, chunks_per_worker)], idx_all)

    def clamp_row(c, carry):
        for j in range(_CHUNK // _LANES):
            s = pl.ds(j * _LANES, _LANES)
            idx_all[c, s] = jnp.minimum(jnp.maximum(idx_all[c, s], 0),
                                        MAX_LEN - 1)
        return carry

    lax.fori_loop(0, chunks_per_worker, clamp_row, 0, unroll=False)

    def start_fetch(c, slot):
        pltpu.async_copy(pe_copy.at[idx_all.at[c]], perows.at[slot],
                         sem_in.at[slot])
        pltpu.async_copy(x_hbm.at[pl.ds(row0 + c * _CHUNK, _CHUNK)],
                         xbuf.at[slot], sem_in.at[slot])

    def wait_fetch(c, slot):
        pltpu.make_async_copy(pe_copy.at[idx_all.at[c]], perows.at[slot],
                              sem_in.at[slot]).wait()
        pltpu.make_async_copy(x_hbm.at[pl.ds(row0 + c * _CHUNK, _CHUNK)],
                              xbuf.at[slot], sem_in.at[slot]).wait()

    def wait_wb(c, slot):
        pltpu.make_async_copy(xbuf.at[slot],
                              out_hbm.at[pl.ds(row0 + c * _CHUNK, _CHUNK)],
                              sem_wb.at[slot]).wait()

    def step(c, slot):
        nslot = (slot + 1) % _DEPTH

        @pl.when(c + 1 < chunks_per_worker)
        def _():
            @pl.when(c + 1 >= _DEPTH)
            def _():
                wait_wb(c + 1 - _DEPTH, nslot)
            start_fetch(c + 1, nslot)

        wait_fetch(c, slot)

        def add_row(r, carry2, slot=slot):
            for j in range(D_MODEL // _LANES):
                s = pl.ds(j * _LANES, _LANES)
                xbuf[slot, r, s] = xbuf[slot, r, s] + perows[slot, r, s]
            return carry2

        lax.fori_loop(0, _CHUNK, add_row, 0, unroll=False)

        pltpu.async_copy(xbuf.at[slot],
                         out_hbm.at[pl.ds(row0 + c * _CHUNK, _CHUNK)],
                         sem_wb.at[slot])

    start_fetch(0, 0)

    main_trips = chunks_per_worker // _DEPTH

    def group_body(g, carry):
        for slot in range(_DEPTH):
            step(g * _DEPTH + slot, slot)
        return carry

    lax.fori_loop(0, main_trips, group_body, 0, unroll=False)
    for c in range(main_trips * _DEPTH, chunks_per_worker):
        step(c, c % _DEPTH)

    for c in range(chunks_per_worker - _DEPTH, chunks_per_worker):
        wait_wb(c, c % _DEPTH)


def kernel(x, timestamps, pe):
    batch, seq, d = x.shape
    n = batch * seq
    assert d == D_MODEL and n % (_NUM_WORKERS * _CHUNK) == 0
    chunks_per_worker = n // (_NUM_WORKERS * _CHUNK)
    assert chunks_per_worker >= 2 * _DEPTH

    x2 = x.reshape(n, d)
    ts2 = timestamps.astype(jnp.int32).reshape(n // _CHUNK, _CHUNK)
    pe_rep = jnp.broadcast_to(pe, (_REPLICAS,) + pe.shape)

    mesh = plsc.VectorSubcoreMesh(core_axis_name="c", subcore_axis_name="s")
    body = functools.partial(_sc_body, chunks_per_worker=chunks_per_worker)
    out = pl.kernel(
        body,
        out_type=jax.ShapeDtypeStruct((n, d), jnp.float32),
        mesh=mesh,
        scratch_types=[
            pltpu.VMEM((chunks_per_worker, _CHUNK), jnp.int32),
            pltpu.VMEM((_DEPTH, _CHUNK, D_MODEL), jnp.float32),
            pltpu.VMEM((_DEPTH, _CHUNK, D_MODEL), jnp.float32),
            pltpu.SemaphoreType.DMA((_DEPTH,)),
            pltpu.SemaphoreType.DMA((_DEPTH,)),
        ],
    )(x2, ts2, pe_rep)
    return out.reshape(batch, seq, d)
